# baseline (speedup 1.0000x reference)
import functools

import jax
import jax.numpy as jnp
from jax import lax
from jax.experimental import pallas as pl
from jax.experimental.pallas import tpu as pltpu

F32 = jnp.float32
HI = lax.Precision.HIGHEST

N = 10000
E = 320000
HID = 128

EDGE_BLK = 3200
NODE_BLK = 2000


def _silu(x):
    return x * jax.nn.sigmoid(x)


def _dot(a, b):
    return jnp.dot(a, b, preferred_element_type=F32, precision=HI)


def _mlp2_body(x_ref, w1_ref, b1_ref, w2_ref, b2_ref, o_ref):
    h = _silu(_dot(x_ref[...], w1_ref[...]) + b1_ref[...])
    o_ref[...] = _dot(h, w2_ref[...]) + b2_ref[...]


def _mlp2(x, w1, b1, w2, b2, blk):
    rows, d_in = x.shape
    d_hid = w1.shape[1]
    d_out = w2.shape[1]
    grid = rows // blk
    return pl.pallas_call(
        _mlp2_body,
        grid=(grid,),
        in_specs=[
            pl.BlockSpec((blk, d_in), lambda i: (i, 0)),
            pl.BlockSpec((d_in, d_hid), lambda i: (0, 0)),
            pl.BlockSpec((1, d_hid), lambda i: (0, 0)),
            pl.BlockSpec((d_hid, d_out), lambda i: (0, 0)),
            pl.BlockSpec((1, d_out), lambda i: (0, 0)),
        ],
        out_specs=pl.BlockSpec((blk, d_out), lambda i: (i, 0)),
        out_shape=jax.ShapeDtypeStruct((rows, d_out), F32),
    )(x, w1, b1.reshape(1, -1), w2, b2.reshape(1, -1))


def _pq_body(hn_ref, ws_ref, wr_ref, b1_ref, p_ref, q_ref):
    hn = hn_ref[...]
    p_ref[...] = _dot(hn, ws_ref[...])
    q_ref[...] = _dot(hn, wr_ref[...]) + b1_ref[...]


def _pq(h_n, w1s, w1r, b1):
    grid = N // NODE_BLK
    return pl.pallas_call(
        _pq_body,
        grid=(grid,),
        in_specs=[
            pl.BlockSpec((NODE_BLK, HID), lambda i: (i, 0)),
            pl.BlockSpec((HID, HID), lambda i: (0, 0)),
            pl.BlockSpec((HID, HID), lambda i: (0, 0)),
            pl.BlockSpec((1, HID), lambda i: (0, 0)),
        ],
        out_specs=[
            pl.BlockSpec((NODE_BLK, HID), lambda i: (i, 0)),
            pl.BlockSpec((NODE_BLK, HID), lambda i: (i, 0)),
        ],
        out_shape=[
            jax.ShapeDtypeStruct((N, HID), F32),
            jax.ShapeDtypeStruct((N, HID), F32),
        ],
    )(h_n, w1s, w1r, b1.reshape(1, -1))


def _edge_body(he_ref, g_ref, w1_ref, w2_ref, b2_ref, ne_ref, heo_ref):
    he = he_ref[...]
    h = _silu(_dot(he, w1_ref[...]) + g_ref[...])
    ne = _dot(h, w2_ref[...]) + b2_ref[...]
    ne_ref[...] = ne
    heo_ref[...] = he + ne


def _edge_step(h_e, g, w1e, w2, b2):
    grid = E // EDGE_BLK
    return pl.pallas_call(
        _edge_body,
        grid=(grid,),
        in_specs=[
            pl.BlockSpec((EDGE_BLK, HID), lambda i: (i, 0)),
            pl.BlockSpec((EDGE_BLK, HID), lambda i: (i, 0)),
            pl.BlockSpec((HID, HID), lambda i: (0, 0)),
            pl.BlockSpec((HID, HID), lambda i: (0, 0)),
            pl.BlockSpec((1, HID), lambda i: (0, 0)),
        ],
        out_specs=[
            pl.BlockSpec((EDGE_BLK, HID), lambda i: (i, 0)),
            pl.BlockSpec((EDGE_BLK, HID), lambda i: (i, 0)),
        ],
        out_shape=[
            jax.ShapeDtypeStruct((E, HID), F32),
            jax.ShapeDtypeStruct((E, HID), F32),
        ],
    )(h_e, g, w1e, w2, b2.reshape(1, -1))


def _node_body(hn_ref, agg_ref, v1a_ref, v1b_ref, c1_ref, v2_ref, c2_ref, o_ref):
    hn = hn_ref[...]
    h = _silu(_dot(hn, v1a_ref[...]) + _dot(agg_ref[...], v1b_ref[...]) + c1_ref[...])
    o_ref[...] = hn + _dot(h, v2_ref[...]) + c2_ref[...]


def _node_step(h_n, agg, v1a, v1b, c1, v2, c2):
    grid = N // NODE_BLK
    return pl.pallas_call(
        _node_body,
        grid=(grid,),
        in_specs=[
            pl.BlockSpec((NODE_BLK, HID), lambda i: (i, 0)),
            pl.BlockSpec((NODE_BLK, HID), lambda i: (i, 0)),
            pl.BlockSpec((HID, HID), lambda i: (0, 0)),
            pl.BlockSpec((HID, HID), lambda i: (0, 0)),
            pl.BlockSpec((1, HID), lambda i: (0, 0)),
            pl.BlockSpec((HID, HID), lambda i: (0, 0)),
            pl.BlockSpec((1, HID), lambda i: (0, 0)),
        ],
        out_specs=pl.BlockSpec((NODE_BLK, HID), lambda i: (i, 0)),
        out_shape=jax.ShapeDtypeStruct((N, HID), F32),
    )(h_n, agg, v1a, v1b, c1.reshape(1, -1), v2, c2.reshape(1, -1))


def kernel(nodes, edges, senders, receivers, params):
    h_n = _mlp2(nodes, params["node_encoder"]["W1"], params["node_encoder"]["b1"],
                params["node_encoder"]["W2"], params["node_encoder"]["b2"], NODE_BLK)
    h_e = _mlp2(edges, params["edge_encoder"]["W1"], params["edge_encoder"]["b1"],
                params["edge_encoder"]["W2"], params["edge_encoder"]["b2"], EDGE_BLK)

    for bp in params["blocks"]:
        ew1 = bp["edge_mlp"]["W1"]
        w1e, w1s, w1r = ew1[:HID], ew1[HID:2 * HID], ew1[2 * HID:]
        p, q = _pq(h_n, w1s, w1r, bp["edge_mlp"]["b1"])
        g = jnp.take(p, senders, axis=0) + jnp.take(q, receivers, axis=0)
        new_e, h_e = _edge_step(h_e, g, w1e, bp["edge_mlp"]["W2"], bp["edge_mlp"]["b2"])
        agg = jax.ops.segment_sum(new_e, receivers, num_segments=N)
        nv1 = bp["node_mlp"]["W1"]
        h_n = _node_step(h_n, agg, nv1[:HID], nv1[HID:], bp["node_mlp"]["b1"],
                         bp["node_mlp"]["W2"], bp["node_mlp"]["b2"])

    dec = params["decoder"]
    w2p = jnp.pad(dec["W2"], ((0, 0), (0, HID - dec["W2"].shape[1])))
    b2p = jnp.pad(dec["b2"], (0, HID - dec["b2"].shape[0]))
    out = _mlp2(h_n, dec["W1"], dec["b1"], w2p, b2p, NODE_BLK)
    return out[:, :dec["W2"].shape[1]]


# speedup vs baseline: 3.4657x; 3.4657x over previous
import functools

import jax
import jax.numpy as jnp
from jax import lax
from jax.experimental import pallas as pl
from jax.experimental.pallas import tpu as pltpu
from jax.experimental.pallas import tpu_sc as plsc

F32 = jnp.float32
HI = lax.Precision.HIGHEST

N = 10000
E = 320000
HID = 128

EDGE_BLK = 3200
NODE_BLK = 2000

NC, NS = 2, 16
NW = NC * NS
EPW = E // NW
CHUNK = 80
KSUB = 5
SUPER = CHUNK * KSUB
NSUPER = EPW // SUPER
NSCT = E // SUPER
NPAD = 10240
NPT = NPAD // NS


def _silu(x):
    return x * jax.nn.sigmoid(x)


def _dot(a, b):
    return jnp.dot(a, b, preferred_element_type=F32)


def _sc_mesh():
    return plsc.VectorSubcoreMesh(core_axis_name="c", subcore_axis_name="s")


def _gather_ps_qr(p, q, s3, r3):
    @functools.partial(
        pl.kernel,
        out_type=[
            jax.ShapeDtypeStruct((E, HID), F32),
            jax.ShapeDtypeStruct((E, HID), F32),
        ],
        mesh=_sc_mesh(),
        scratch_types=[
            pltpu.VMEM((KSUB, CHUNK), jnp.int32),
            pltpu.VMEM((KSUB, CHUNK), jnp.int32),
            pltpu.VMEM((SUPER, HID), F32),
            pltpu.VMEM((SUPER, HID), F32),
            pltpu.SemaphoreType.DMA,
            pltpu.SemaphoreType.DMA,
        ],
    )
    def k(p_hbm, q_hbm, s_hbm, r_hbm, ps_hbm, qr_hbm,
          sidx, ridx, psbuf, qrbuf, gsem, wsem):
        wid = lax.axis_index("s") * NC + lax.axis_index("c")
        base = wid * NSUPER

        def body(g, carry):
            sc = base + g
            eoff = sc * SUPER
            pltpu.sync_copy(s_hbm.at[sc], sidx)
            pltpu.sync_copy(r_hbm.at[sc], ridx)
            copies = []
            for j in range(KSUB):
                copies.append(pltpu.async_copy(
                    p_hbm.at[sidx.at[j]],
                    psbuf.at[pl.ds(j * CHUNK, CHUNK)], gsem))
                copies.append(pltpu.async_copy(
                    q_hbm.at[ridx.at[j]],
                    qrbuf.at[pl.ds(j * CHUNK, CHUNK)], gsem))
            for c in copies:
                c.wait()
            w1 = pltpu.async_copy(psbuf, ps_hbm.at[pl.ds(eoff, SUPER)], wsem)
            w2 = pltpu.async_copy(qrbuf, qr_hbm.at[pl.ds(eoff, SUPER)], wsem)
            w1.wait()
            w2.wait()
            return carry

        lax.fori_loop(0, NSUPER, body, 0)

    return k(p, q, s3, r3)


def _scatter_agg(new_e, r3, zeros):
    @functools.partial(
        pl.kernel,
        out_type=jax.ShapeDtypeStruct((NC * NPAD, HID), F32),
        mesh=_sc_mesh(),
        scratch_types=[
            pltpu.VMEM_SHARED((NPAD, HID), F32),
            pltpu.VMEM((KSUB, CHUNK), jnp.int32),
            pltpu.VMEM((CHUNK, HID), F32),
        ],
    )
    def k(ne_hbm, r_hbm, z_hbm, agg_hbm, aggs, ridx, rows):
        cid = lax.axis_index("c")
        sid = lax.axis_index("s")
        wid = sid * NC + cid
        base = wid * NSUPER

        pltpu.sync_copy(z_hbm.at[pl.ds(sid * NPT, NPT)],
                        aggs.at[pl.ds(sid * NPT, NPT)])
        plsc.subcore_barrier()

        def body(g, carry):
            sc = base + g
            eoff = sc * SUPER
            pltpu.sync_copy(r_hbm.at[sc], ridx)
            for j in range(KSUB):
                pltpu.sync_copy(ne_hbm.at[pl.ds(eoff + j * CHUNK, CHUNK)],
                                rows)
                pltpu.sync_copy(rows, aggs.at[ridx.at[j]], add=True)
            return carry

        lax.fori_loop(0, NSUPER, body, 0)
        plsc.subcore_barrier()
        pltpu.sync_copy(aggs.at[pl.ds(sid * NPT, NPT)],
                        agg_hbm.at[pl.ds(cid * NPAD + sid * NPT, NPT)])

    return k(new_e, r3, zeros)


def _mlp2_body(x_ref, w1_ref, b1_ref, w2_ref, b2_ref, o_ref):
    h = _silu(_dot(x_ref[...], w1_ref[...]) + b1_ref[...])
    o_ref[...] = _dot(h, w2_ref[...]) + b2_ref[...]


def _mlp2(x, w1, b1, w2, b2, blk):
    rows, d_in = x.shape
    d_hid = w1.shape[1]
    d_out = w2.shape[1]
    grid = rows // blk
    return pl.pallas_call(
        _mlp2_body,
        grid=(grid,),
        in_specs=[
            pl.BlockSpec((blk, d_in), lambda i: (i, 0)),
            pl.BlockSpec((d_in, d_hid), lambda i: (0, 0)),
            pl.BlockSpec((1, d_hid), lambda i: (0, 0)),
            pl.BlockSpec((d_hid, d_out), lambda i: (0, 0)),
            pl.BlockSpec((1, d_out), lambda i: (0, 0)),
        ],
        out_specs=pl.BlockSpec((blk, d_out), lambda i: (i, 0)),
        out_shape=jax.ShapeDtypeStruct((rows, d_out), F32),
    )(x, w1, b1.reshape(1, -1), w2, b2.reshape(1, -1))


def _pq_body(hn_ref, ws_ref, wr_ref, b1_ref, p_ref, q_ref):
    hn = hn_ref[...]
    p_ref[...] = _dot(hn, ws_ref[...])
    q_ref[...] = _dot(hn, wr_ref[...]) + b1_ref[...]


def _pq(h_n, w1s, w1r, b1):
    grid = N // NODE_BLK
    return pl.pallas_call(
        _pq_body,
        grid=(grid,),
        in_specs=[
            pl.BlockSpec((NODE_BLK, HID), lambda i: (i, 0)),
            pl.BlockSpec((HID, HID), lambda i: (0, 0)),
            pl.BlockSpec((HID, HID), lambda i: (0, 0)),
            pl.BlockSpec((1, HID), lambda i: (0, 0)),
        ],
        out_specs=[
            pl.BlockSpec((NODE_BLK, HID), lambda i: (i, 0)),
            pl.BlockSpec((NODE_BLK, HID), lambda i: (i, 0)),
        ],
        out_shape=[
            jax.ShapeDtypeStruct((N, HID), F32),
            jax.ShapeDtypeStruct((N, HID), F32),
        ],
    )(h_n, w1s, w1r, b1.reshape(1, -1))


def _edge_body(he_ref, ps_ref, qr_ref, w1_ref, w2_ref, b2_ref, ne_ref, heo_ref):
    he = he_ref[...]
    h = _silu(_dot(he, w1_ref[...]) + ps_ref[...] + qr_ref[...])
    ne = _dot(h, w2_ref[...]) + b2_ref[...]
    ne_ref[...] = ne
    heo_ref[...] = he + ne


def _edge_step(h_e, ps, qr, w1e, w2, b2):
    grid = E // EDGE_BLK
    return pl.pallas_call(
        _edge_body,
        grid=(grid,),
        in_specs=[
            pl.BlockSpec((EDGE_BLK, HID), lambda i: (i, 0)),
            pl.BlockSpec((EDGE_BLK, HID), lambda i: (i, 0)),
            pl.BlockSpec((EDGE_BLK, HID), lambda i: (i, 0)),
            pl.BlockSpec((HID, HID), lambda i: (0, 0)),
            pl.BlockSpec((HID, HID), lambda i: (0, 0)),
            pl.BlockSpec((1, HID), lambda i: (0, 0)),
        ],
        out_specs=[
            pl.BlockSpec((EDGE_BLK, HID), lambda i: (i, 0)),
            pl.BlockSpec((EDGE_BLK, HID), lambda i: (i, 0)),
        ],
        out_shape=[
            jax.ShapeDtypeStruct((E, HID), F32),
            jax.ShapeDtypeStruct((E, HID), F32),
        ],
    )(h_e, ps, qr, w1e, w2, b2.reshape(1, -1))


def _node_body(hn_ref, a0_ref, a1_ref, v1a_ref, v1b_ref, c1_ref, v2_ref,
               c2_ref, o_ref):
    hn = hn_ref[...]
    agg = a0_ref[...] + a1_ref[...]
    h = _silu(_dot(hn, v1a_ref[...]) + _dot(agg, v1b_ref[...]) + c1_ref[...])
    o_ref[...] = hn + _dot(h, v2_ref[...]) + c2_ref[...]


def _node_step(h_n, agg2, v1a, v1b, c1, v2, c2):
    grid = N // NODE_BLK
    return pl.pallas_call(
        _node_body,
        grid=(grid,),
        in_specs=[
            pl.BlockSpec((NODE_BLK, HID), lambda i: (i, 0)),
            pl.BlockSpec((NODE_BLK, HID), lambda i: (i, 0)),
            pl.BlockSpec((NODE_BLK, HID), lambda i: (i, 0)),
            pl.BlockSpec((HID, HID), lambda i: (0, 0)),
            pl.BlockSpec((HID, HID), lambda i: (0, 0)),
            pl.BlockSpec((1, HID), lambda i: (0, 0)),
            pl.BlockSpec((HID, HID), lambda i: (0, 0)),
            pl.BlockSpec((1, HID), lambda i: (0, 0)),
        ],
        out_specs=pl.BlockSpec((NODE_BLK, HID), lambda i: (i, 0)),
        out_shape=jax.ShapeDtypeStruct((N, HID), F32),
    )(h_n, agg2[:N], agg2[NPAD:NPAD + N], v1a, v1b, c1.reshape(1, -1), v2,
      c2.reshape(1, -1))


def kernel(nodes, edges, senders, receivers, params):
    s3 = senders.reshape(NSCT, KSUB, CHUNK)
    r3 = receivers.reshape(NSCT, KSUB, CHUNK)
    zeros = jnp.zeros((NPAD, HID), F32)

    h_n = _mlp2(nodes, params["node_encoder"]["W1"], params["node_encoder"]["b1"],
                params["node_encoder"]["W2"], params["node_encoder"]["b2"], NODE_BLK)
    h_e = _mlp2(edges, params["edge_encoder"]["W1"], params["edge_encoder"]["b1"],
                params["edge_encoder"]["W2"], params["edge_encoder"]["b2"], EDGE_BLK)

    for bp in params["blocks"]:
        ew1 = bp["edge_mlp"]["W1"]
        w1e, w1s, w1r = ew1[:HID], ew1[HID:2 * HID], ew1[2 * HID:]
        p, q = _pq(h_n, w1s, w1r, bp["edge_mlp"]["b1"])
        ps, qr = _gather_ps_qr(p, q, s3, r3)
        new_e, h_e = _edge_step(h_e, ps, qr, w1e, bp["edge_mlp"]["W2"],
                                bp["edge_mlp"]["b2"])
        agg2 = _scatter_agg(new_e, r3, zeros)
        nv1 = bp["node_mlp"]["W1"]
        h_n = _node_step(h_n, agg2, nv1[:HID], nv1[HID:], bp["node_mlp"]["b1"],
                         bp["node_mlp"]["W2"], bp["node_mlp"]["b2"])

    dec = params["decoder"]
    w2p = jnp.pad(dec["W2"], ((0, 0), (0, HID - dec["W2"].shape[1])))
    b2p = jnp.pad(dec["b2"], (0, HID - dec["b2"].shape[0]))
    out = _mlp2(h_n, dec["W1"], dec["b1"], w2p, b2p, NODE_BLK)
    return out[:, :dec["W2"].shape[1]]


# speedup vs baseline: 3.6625x; 1.0568x over previous
import functools

import jax
import jax.numpy as jnp
from jax import lax
from jax.experimental import pallas as pl
from jax.experimental.pallas import tpu as pltpu
from jax.experimental.pallas import tpu_sc as plsc

F32 = jnp.float32
HI = lax.Precision.HIGHEST

N = 10000
E = 320000
HID = 128

EDGE_BLK = 3200
NODE_BLK = 2000

NC, NS = 2, 16
NW = NC * NS
EPW = E // NW
CHUNK = 80
KSUB = 5
SUPER = CHUNK * KSUB
NSUPER = EPW // SUPER
NSCT = E // SUPER
NPAD = 10240
NPT = NPAD // NS


def _silu(x):
    return x * jax.nn.sigmoid(x)


def _dot(a, b):
    return jnp.dot(a, b, preferred_element_type=F32)


def _sc_mesh():
    return plsc.VectorSubcoreMesh(core_axis_name="c", subcore_axis_name="s")


def _gather_ps_qr(p, q, s3, r3):
    @functools.partial(
        pl.kernel,
        out_type=[
            jax.ShapeDtypeStruct((E, HID), F32),
            jax.ShapeDtypeStruct((E, HID), F32),
        ],
        mesh=_sc_mesh(),
        scratch_types=[
            pltpu.VMEM((KSUB, CHUNK), jnp.int32),
            pltpu.VMEM((KSUB, CHUNK), jnp.int32),
            pltpu.VMEM((SUPER, HID), F32),
            pltpu.VMEM((SUPER, HID), F32),
            pltpu.SemaphoreType.DMA,
            pltpu.SemaphoreType.DMA,
        ],
    )
    def k(p_hbm, q_hbm, s_hbm, r_hbm, ps_hbm, qr_hbm,
          sidx, ridx, psbuf, qrbuf, gsem, wsem):
        wid = lax.axis_index("s") * NC + lax.axis_index("c")
        base = wid * NSUPER

        def body(g, carry):
            sc = base + g
            eoff = sc * SUPER
            pltpu.sync_copy(s_hbm.at[sc], sidx)
            pltpu.sync_copy(r_hbm.at[sc], ridx)
            copies = []
            for j in range(KSUB):
                copies.append(pltpu.async_copy(
                    p_hbm.at[sidx.at[j]],
                    psbuf.at[pl.ds(j * CHUNK, CHUNK)], gsem))
                copies.append(pltpu.async_copy(
                    q_hbm.at[ridx.at[j]],
                    qrbuf.at[pl.ds(j * CHUNK, CHUNK)], gsem))
            for c in copies:
                c.wait()
            w1 = pltpu.async_copy(psbuf, ps_hbm.at[pl.ds(eoff, SUPER)], wsem)
            w2 = pltpu.async_copy(qrbuf, qr_hbm.at[pl.ds(eoff, SUPER)], wsem)
            w1.wait()
            w2.wait()
            return carry

        lax.fori_loop(0, NSUPER, body, 0)

    return k(p, q, s3, r3)


def _scatter_agg(new_e, r3, zeros):
    @functools.partial(
        pl.kernel,
        out_type=jax.ShapeDtypeStruct((NC * NPAD, HID), F32),
        mesh=_sc_mesh(),
        scratch_types=[
            pltpu.VMEM_SHARED((NPAD, HID), F32),
            pltpu.VMEM((KSUB, CHUNK), jnp.int32),
            pltpu.VMEM((2, CHUNK, HID), F32),
            pltpu.SemaphoreType.DMA,
            pltpu.SemaphoreType.DMA,
        ],
    )
    def k(ne_hbm, r_hbm, z_hbm, agg_hbm, aggs, ridx, rows, sem0, sem1):
        cid = lax.axis_index("c")
        sid = lax.axis_index("s")
        wid = sid * NC + cid
        base = wid * NSUPER
        sems = (sem0, sem1)

        pltpu.sync_copy(z_hbm.at[pl.ds(sid * NPT, NPT)],
                        aggs.at[pl.ds(sid * NPT, NPT)])
        plsc.subcore_barrier()

        def body(g, carry):
            sc = base + g
            eoff = sc * SUPER
            pltpu.sync_copy(r_hbm.at[sc], ridx)
            descs = [pltpu.async_copy(ne_hbm.at[pl.ds(eoff, CHUNK)],
                                      rows.at[0], sems[0])]
            for j in range(KSUB):
                b = j % 2
                descs[j].wait()
                if j + 1 < KSUB:
                    nb = (j + 1) % 2
                    descs.append(pltpu.async_copy(
                        ne_hbm.at[pl.ds(eoff + (j + 1) * CHUNK, CHUNK)],
                        rows.at[nb], sems[nb]))
                pltpu.sync_copy(rows.at[b], aggs.at[ridx.at[j]], add=True)
            return carry

        lax.fori_loop(0, NSUPER, body, 0)
        plsc.subcore_barrier()
        pltpu.sync_copy(aggs.at[pl.ds(sid * NPT, NPT)],
                        agg_hbm.at[pl.ds(cid * NPAD + sid * NPT, NPT)])

    return k(new_e, r3, zeros)


def _mlp2_body(x_ref, w1_ref, b1_ref, w2_ref, b2_ref, o_ref):
    h = _silu(_dot(x_ref[...], w1_ref[...]) + b1_ref[...])
    o_ref[...] = _dot(h, w2_ref[...]) + b2_ref[...]


def _mlp2(x, w1, b1, w2, b2, blk):
    rows, d_in = x.shape
    d_hid = w1.shape[1]
    d_out = w2.shape[1]
    grid = rows // blk
    return pl.pallas_call(
        _mlp2_body,
        grid=(grid,),
        in_specs=[
            pl.BlockSpec((blk, d_in), lambda i: (i, 0)),
            pl.BlockSpec((d_in, d_hid), lambda i: (0, 0)),
            pl.BlockSpec((1, d_hid), lambda i: (0, 0)),
            pl.BlockSpec((d_hid, d_out), lambda i: (0, 0)),
            pl.BlockSpec((1, d_out), lambda i: (0, 0)),
        ],
        out_specs=pl.BlockSpec((blk, d_out), lambda i: (i, 0)),
        out_shape=jax.ShapeDtypeStruct((rows, d_out), F32),
    )(x, w1, b1.reshape(1, -1), w2, b2.reshape(1, -1))


def _pq_body(hn_ref, ws_ref, wr_ref, b1_ref, p_ref, q_ref):
    hn = hn_ref[...]
    p_ref[...] = _dot(hn, ws_ref[...])
    q_ref[...] = _dot(hn, wr_ref[...]) + b1_ref[...]


def _pq(h_n, w1s, w1r, b1):
    grid = N // NODE_BLK
    return pl.pallas_call(
        _pq_body,
        grid=(grid,),
        in_specs=[
            pl.BlockSpec((NODE_BLK, HID), lambda i: (i, 0)),
            pl.BlockSpec((HID, HID), lambda i: (0, 0)),
            pl.BlockSpec((HID, HID), lambda i: (0, 0)),
            pl.BlockSpec((1, HID), lambda i: (0, 0)),
        ],
        out_specs=[
            pl.BlockSpec((NODE_BLK, HID), lambda i: (i, 0)),
            pl.BlockSpec((NODE_BLK, HID), lambda i: (i, 0)),
        ],
        out_shape=[
            jax.ShapeDtypeStruct((N, HID), F32),
            jax.ShapeDtypeStruct((N, HID), F32),
        ],
    )(h_n, w1s, w1r, b1.reshape(1, -1))


def _edge_body(he_ref, ps_ref, qr_ref, w1_ref, w2_ref, b2_ref, ne_ref, heo_ref):
    he = he_ref[...]
    h = _silu(_dot(he, w1_ref[...]) + ps_ref[...] + qr_ref[...])
    ne = _dot(h, w2_ref[...]) + b2_ref[...]
    ne_ref[...] = ne
    heo_ref[...] = he + ne


def _edge_step(h_e, ps, qr, w1e, w2, b2):
    grid = E // EDGE_BLK
    return pl.pallas_call(
        _edge_body,
        grid=(grid,),
        in_specs=[
            pl.BlockSpec((EDGE_BLK, HID), lambda i: (i, 0)),
            pl.BlockSpec((EDGE_BLK, HID), lambda i: (i, 0)),
            pl.BlockSpec((EDGE_BLK, HID), lambda i: (i, 0)),
            pl.BlockSpec((HID, HID), lambda i: (0, 0)),
            pl.BlockSpec((HID, HID), lambda i: (0, 0)),
            pl.BlockSpec((1, HID), lambda i: (0, 0)),
        ],
        out_specs=[
            pl.BlockSpec((EDGE_BLK, HID), lambda i: (i, 0)),
            pl.BlockSpec((EDGE_BLK, HID), lambda i: (i, 0)),
        ],
        out_shape=[
            jax.ShapeDtypeStruct((E, HID), F32),
            jax.ShapeDtypeStruct((E, HID), F32),
        ],
    )(h_e, ps, qr, w1e, w2, b2.reshape(1, -1))


def _node_body(hn_ref, a0_ref, a1_ref, v1a_ref, v1b_ref, c1_ref, v2_ref,
               c2_ref, o_ref):
    hn = hn_ref[...]
    agg = a0_ref[...] + a1_ref[...]
    h = _silu(_dot(hn, v1a_ref[...]) + _dot(agg, v1b_ref[...]) + c1_ref[...])
    o_ref[...] = hn + _dot(h, v2_ref[...]) + c2_ref[...]


def _node_step(h_n, agg2, v1a, v1b, c1, v2, c2):
    grid = N // NODE_BLK
    return pl.pallas_call(
        _node_body,
        grid=(grid,),
        in_specs=[
            pl.BlockSpec((NODE_BLK, HID), lambda i: (i, 0)),
            pl.BlockSpec((NODE_BLK, HID), lambda i: (i, 0)),
            pl.BlockSpec((NODE_BLK, HID), lambda i: (i, 0)),
            pl.BlockSpec((HID, HID), lambda i: (0, 0)),
            pl.BlockSpec((HID, HID), lambda i: (0, 0)),
            pl.BlockSpec((1, HID), lambda i: (0, 0)),
            pl.BlockSpec((HID, HID), lambda i: (0, 0)),
            pl.BlockSpec((1, HID), lambda i: (0, 0)),
        ],
        out_specs=pl.BlockSpec((NODE_BLK, HID), lambda i: (i, 0)),
        out_shape=jax.ShapeDtypeStruct((N, HID), F32),
    )(h_n, agg2[:N], agg2[NPAD:NPAD + N], v1a, v1b, c1.reshape(1, -1), v2,
      c2.reshape(1, -1))


def kernel(nodes, edges, senders, receivers, params):
    s3 = senders.reshape(NSCT, KSUB, CHUNK)
    r3 = receivers.reshape(NSCT, KSUB, CHUNK)
    zeros = jnp.zeros((NPAD, HID), F32)

    h_n = _mlp2(nodes, params["node_encoder"]["W1"], params["node_encoder"]["b1"],
                params["node_encoder"]["W2"], params["node_encoder"]["b2"], NODE_BLK)
    h_e = _mlp2(edges, params["edge_encoder"]["W1"], params["edge_encoder"]["b1"],
                params["edge_encoder"]["W2"], params["edge_encoder"]["b2"], EDGE_BLK)

    for bp in params["blocks"]:
        ew1 = bp["edge_mlp"]["W1"]
        w1e, w1s, w1r = ew1[:HID], ew1[HID:2 * HID], ew1[2 * HID:]
        p, q = _pq(h_n, w1s, w1r, bp["edge_mlp"]["b1"])
        ps, qr = _gather_ps_qr(p, q, s3, r3)
        new_e, h_e = _edge_step(h_e, ps, qr, w1e, bp["edge_mlp"]["W2"],
                                bp["edge_mlp"]["b2"])
        agg2 = _scatter_agg(new_e, r3, zeros)
        nv1 = bp["node_mlp"]["W1"]
        h_n = _node_step(h_n, agg2, nv1[:HID], nv1[HID:], bp["node_mlp"]["b1"],
                         bp["node_mlp"]["W2"], bp["node_mlp"]["b2"])

    dec = params["decoder"]
    w2p = jnp.pad(dec["W2"], ((0, 0), (0, HID - dec["W2"].shape[1])))
    b2p = jnp.pad(dec["b2"], (0, HID - dec["b2"].shape[0]))
    out = _mlp2(h_n, dec["W1"], dec["b1"], w2p, b2p, NODE_BLK)
    return out[:, :dec["W2"].shape[1]]
